# baseline (speedup 1.0000x reference)
import functools

import jax
import jax.numpy as jnp
from jax import lax
from jax.experimental import pallas as pl
from jax.experimental.pallas import tpu as pltpu
from jax.experimental.pallas import tpu_sc as plsc

B = 4096
L = 200
D = 64
V = 1_000_000
SCALE = 8.0
PITCH = 135

_info = plsc.get_sparse_core_info()
_NC, _NS = _info.num_cores, _info.num_subcores
NW = _NC * _NS
CHUNK = 128
NBUF = 3
NFULL = (L // NBUF) * NBUF

_mesh = plsc.VectorSubcoreMesh(core_axis_name="c", subcore_axis_name="s")


@functools.partial(
    pl.kernel,
    mesh=_mesh,
    out_type=jax.ShapeDtypeStruct((L, D, B), jnp.float32),
    scratch_types=[
        pltpu.VMEM((L, CHUNK), jnp.int32),
        pltpu.VMEM((NBUF, CHUNK), jnp.int32),
        pltpu.VMEM((CHUNK, 128), jnp.float32),
        pltpu.VMEM((CHUNK, 128), jnp.float32),
        pltpu.VMEM((CHUNK, 128), jnp.float32),
        pltpu.VMEM((D, PITCH), jnp.float32),
        pltpu.VMEM((D, CHUNK), jnp.float32),
        pltpu.VMEM((D, CHUNK), jnp.float32),
        pltpu.VMEM((D, CHUNK), jnp.float32),
    ] + [pltpu.SemaphoreType.DMA] * (2 * NBUF),
    compiler_params=pltpu.CompilerParams(
        use_tc_tiling_on_sc=True, needs_layout_passes=False),
)
def _emb(xt_hbm, tab_hbm, out_hbm, idx_v, idx2_v,
         g0, g1, g2, tskew, t0, t1, t2, *sems):
    g_bufs = (g0, g1, g2)
    t_bufs = (t0, t1, t2)
    sem_g = sems[:NBUF]
    sem_o = sems[NBUF:]
    wid = lax.axis_index("s") * _NC + lax.axis_index("c")
    pltpu.sync_copy(xt_hbm.at[:, pl.ds(wid * CHUNK, CHUNK)], idx_v)

    iota = lax.iota(jnp.int32, 16)
    fcol = [f0 + iota for f0 in range(0, D, 16)]

    def _halve(s, b):
        for j in range(8):
            idx2_v[b, pl.ds(j * 16, 16)] = (
                idx_v[s, pl.ds(j * 16, 16)] >> 1)

    for b in range(NBUF):
        _halve(b, b)
        pltpu.async_copy(tab_hbm.at[idx2_v.at[b]], g_bufs[b], sem_g[b])

    def unit(s, b, warm, fire):
        g_b = g_bufs[b]
        t_b = t_bufs[b]
        pltpu.make_async_copy(
            tab_hbm.at[idx2_v.at[b]], g_b, sem_g[b]).wait()

        @pl.when(jnp.asarray(warm))
        def _wait_store():
            pltpu.make_async_copy(
                t_b,
                out_hbm.at[0, :, pl.ds(wid * CHUNK, CHUNK)],
                sem_o[b]).wait()

        @plsc.parallel_loop(0, CHUNK // 16)
        def _tr(j):
            parvec = (idx_v[s, pl.ds(j * 16, 16)] & 1) * D
            for rr in range(16):
                half = parvec[rr]
                r = j * 16 + rr
                rvec = jnp.full((16,), r, jnp.int32)
                for k in range(D // 16):
                    vals = g_b[r, pl.ds(half + k * 16, 16)]
                    plsc.store_scatter(tskew, [fcol[k], rvec], vals)

        @plsc.parallel_loop(0, D, unroll=2)
        def _pack(f):
            for j in range(8):
                t_b[f, pl.ds(j * 16, 16)] = (
                    tskew[f, pl.ds(j * 16, 16)] * SCALE)

        pltpu.async_copy(
            t_b,
            out_hbm.at[s, :, pl.ds(wid * CHUNK, CHUNK)],
            sem_o[b])

        if fire:
            @pl.when(jnp.asarray(s + NBUF < L))
            def _fire_gather():
                _halve(s + NBUF, b)
                pltpu.async_copy(
                    tab_hbm.at[idx2_v.at[b]], g_b, sem_g[b])

    def outer(i, carry):
        for b in range(NBUF):
            unit(i * NBUF + b, b, i > 0, True)
        return carry

    lax.fori_loop(0, NFULL // NBUF, outer, 0)
    for e in range(NFULL, L):
        unit(e, e % NBUF, True, False)

    for b in range(NBUF):
        pltpu.make_async_copy(
            t_bufs[b],
            out_hbm.at[0, :, pl.ds(wid * CHUNK, CHUNK)],
            sem_o[b]).wait()


def kernel(x, table):
    tabp = table.reshape(V // 2, 128)
    op = _emb(x.T, tabp)
    return op.transpose(2, 0, 1)


# speedup vs baseline: 1.5745x; 1.5745x over previous
import functools

import jax
import jax.numpy as jnp
from jax import lax
from jax.experimental import pallas as pl
from jax.experimental.pallas import tpu as pltpu
from jax.experimental.pallas import tpu_sc as plsc

B = 4096
L = 200
D = 64
V = 1_000_000
SCALE = 8.0
PITCH = 135

_info = plsc.get_sparse_core_info()
_NC, _NS = _info.num_cores, _info.num_subcores
NW = _NC * _NS
CHUNK = 128
NBUF = 3
NFULL = (L // NBUF) * NBUF

_mesh = plsc.VectorSubcoreMesh(core_axis_name="c", subcore_axis_name="s")


@functools.partial(
    pl.kernel,
    mesh=_mesh,
    out_type=jax.ShapeDtypeStruct((L, D, B), jnp.float32),
    scratch_types=[
        pltpu.VMEM((L, CHUNK), jnp.int32),
        pltpu.VMEM((NBUF, CHUNK), jnp.int32),
        pltpu.VMEM((CHUNK, 128), jnp.float32),
        pltpu.VMEM((CHUNK, 128), jnp.float32),
        pltpu.VMEM((CHUNK, 128), jnp.float32),
        pltpu.VMEM((CHUNK, 128), jnp.float32),
        pltpu.VMEM((D, CHUNK), jnp.float32),
        pltpu.VMEM((D, CHUNK), jnp.float32),
        pltpu.VMEM((D, CHUNK), jnp.float32),
    ] + [pltpu.SemaphoreType.DMA] * (2 * NBUF),
    compiler_params=pltpu.CompilerParams(
        use_tc_tiling_on_sc=True, needs_layout_passes=False),
)
def _emb(xt_hbm, tab_hbm, out_hbm, idx_v, idx2_v,
         g0, g1, g2, gsk, t0, t1, t2, *sems):
    g_bufs = (g0, g1, g2)
    t_bufs = (t0, t1, t2)
    sem_g = sems[:NBUF]
    sem_o = sems[NBUF:]
    wid = lax.axis_index("s") * _NC + lax.axis_index("c")
    pltpu.sync_copy(xt_hbm.at[:, pl.ds(wid * CHUNK, CHUNK)], idx_v)

    iota = lax.iota(jnp.int32, 16)
    rj = [j * 16 + iota for j in range(8)]

    def _halve(s, b):
        for j in range(8):
            idx2_v[b, pl.ds(j * 16, 16)] = (
                idx_v[s, pl.ds(j * 16, 16)] >> 1)

    for b in range(NBUF):
        _halve(b, b)
        pltpu.async_copy(tab_hbm.at[idx2_v.at[b]], g_bufs[b], sem_g[b])

    def unit(s, b, warm, fire):
        g_b = g_bufs[b]
        t_b = t_bufs[b]
        pltpu.make_async_copy(
            tab_hbm.at[idx2_v.at[b]], g_b, sem_g[b]).wait()

        @pl.when(jnp.asarray(warm))
        def _wait_store():
            pltpu.make_async_copy(
                t_b,
                out_hbm.at[0, :, pl.ds(wid * CHUNK, CHUNK)],
                sem_o[b]).wait()

        pj = [(idx_v[s, pl.ds(j * 16, 16)] & 1) * D for j in range(8)]

        for rot in range(16):
            perm = lax.rem(iota + (16 - rot), 16)

            @plsc.parallel_loop(0, CHUNK // 16)
            def _rot(rr):
                r = rr * 16 + rot
                for k in range(8):
                    v = g_b[r, pl.ds(k * 16, 16)]
                    gsk[r, pl.ds(k * 16, 16)] = v.at[perm].get(
                        mode="promise_in_bounds")

        @plsc.parallel_loop(0, D, unroll=2)
        def _tr(f):
            fl = lax.rem(f, 16)
            fg = f - fl
            mvec = lax.rem(iota + fl, 16)
            for j in range(8):
                vals = plsc.load_gather(gsk, [rj[j], pj[j] + fg + mvec])
                t_b[f, pl.ds(j * 16, 16)] = vals * SCALE

        pltpu.async_copy(
            t_b,
            out_hbm.at[s, :, pl.ds(wid * CHUNK, CHUNK)],
            sem_o[b])

        if fire:
            @pl.when(jnp.asarray(s + NBUF < L))
            def _fire_gather():
                _halve(s + NBUF, b)
                pltpu.async_copy(
                    tab_hbm.at[idx2_v.at[b]], g_b, sem_g[b])

    def outer(i, carry):
        for b in range(NBUF):
            unit(i * NBUF + b, b, i > 0, True)
        return carry

    lax.fori_loop(0, NFULL // NBUF, outer, 0)
    for e in range(NFULL, L):
        unit(e, e % NBUF, True, False)

    for b in range(NBUF):
        pltpu.make_async_copy(
            t_bufs[b],
            out_hbm.at[0, :, pl.ds(wid * CHUNK, CHUNK)],
            sem_o[b]).wait()


def kernel(x, table):
    tabp = table.reshape(V // 2, 128)
    op = _emb(x.T, tabp)
    return op.transpose(2, 0, 1)
